# Initial kernel scaffold; baseline (speedup 1.0000x reference)
import jax
import jax.numpy as jnp
from jax.experimental import pallas as pl


def kernel(action_idx, edge_index, bn_row, bn_col, need_q_for_all, input_features, input_feature_s, W1, W2, W3, Wl, bl, W4, W5):
    raise NotImplementedError("write your pallas kernel here")


# speedup vs baseline: 4.4750x; 4.4750x over previous
import functools

import jax
import jax.numpy as jnp
from jax import lax
from jax.experimental import pallas as pl
from jax.experimental.pallas import tpu as pltpu
from jax.experimental.pallas import tpu_sc as plsc


def _enc_body(x_ref, w_ref, o_ref):
    r = jnp.dot(x_ref[...], w_ref[...], preferred_element_type=jnp.float32)
    r = jnp.maximum(r, 0.0)
    n = jnp.sqrt(jnp.sum(r * r, axis=1, keepdims=True))
    o_ref[...] = r / jnp.maximum(n, 1e-12)


def _encode(x, w, blk):
    n, d = x.shape
    emb = w.shape[1]
    return pl.pallas_call(
        _enc_body,
        grid=(n // blk,),
        in_specs=[
            pl.BlockSpec((blk, d), lambda i: (i, 0)),
            pl.BlockSpec((d, emb), lambda i: (0, 0)),
        ],
        out_specs=pl.BlockSpec((blk, emb), lambda i: (i, 0)),
        out_shape=jax.ShapeDtypeStruct((n, emb), jnp.float32),
    )(x, w)


def _wab_body(w2_ref, w3_ref, wl_ref, wa_ref, wb_ref):
    emb = w2_ref.shape[0]
    wl = wl_ref[...]
    dn = (((1,), (1,)), ((), ()))
    wa_ref[...] = lax.dot_general(
        w2_ref[...], wl[:, :emb], dn, preferred_element_type=jnp.float32)
    wb_ref[...] = lax.dot_general(
        w3_ref[...], wl[:, emb:], dn, preferred_element_type=jnp.float32)


def _fold_weights(w2, w3, wl):
    emb = w2.shape[0]
    return pl.pallas_call(
        _wab_body,
        out_shape=(
            jax.ShapeDtypeStruct((emb, emb), jnp.float32),
            jax.ShapeDtypeStruct((emb, emb), jnp.float32),
        ),
    )(w2, w3, wl)


def _layer_body(h_ref, p0_ref, p1_ref, wa_ref, wb_ref, bl_ref, o_ref):
    agg = p0_ref[...] + p1_ref[...]
    y = jnp.dot(h_ref[...], wa_ref[...], preferred_element_type=jnp.float32)
    y = y + jnp.dot(agg, wb_ref[...], preferred_element_type=jnp.float32)
    y = jnp.maximum(y + bl_ref[...], 0.0)
    n = jnp.sqrt(jnp.sum(y * y, axis=1, keepdims=True))
    o_ref[...] = y / jnp.maximum(n, 1e-12)


def _layer(h, p0, p1, wa, wb, bl2, blk):
    n, emb = h.shape
    return pl.pallas_call(
        _layer_body,
        grid=(n // blk,),
        in_specs=[
            pl.BlockSpec((blk, emb), lambda i: (i, 0)),
            pl.BlockSpec((blk, emb), lambda i: (i, 0)),
            pl.BlockSpec((blk, emb), lambda i: (i, 0)),
            pl.BlockSpec((emb, emb), lambda i: (0, 0)),
            pl.BlockSpec((emb, emb), lambda i: (0, 0)),
            pl.BlockSpec((1, emb), lambda i: (0, 0)),
        ],
        out_specs=pl.BlockSpec((blk, emb), lambda i: (i, 0)),
        out_shape=jax.ShapeDtypeStruct((n, emb), jnp.float32),
    )(h, p0, p1, wa, wb, bl2)


def _dec_body(ah_ref, ap0_ref, ap1_ref, hs_ref, ps0_ref, ps1_ref,
              wa_ref, wb_ref, bl_ref, w4_ref, w5_ref, q_ref):
    wa = wa_ref[...]
    wb = wb_ref[...]
    bl = bl_ref[...]

    za = jnp.dot(ah_ref[...], wa, preferred_element_type=jnp.float32)
    za = za + jnp.dot(ap0_ref[...] + ap1_ref[...], wb,
                      preferred_element_type=jnp.float32)
    za = jnp.maximum(za + bl, 0.0)
    na = jnp.sqrt(jnp.sum(za * za, axis=1, keepdims=True))
    za = za / jnp.maximum(na, 1e-12)

    zs = jnp.dot(hs_ref[...], wa, preferred_element_type=jnp.float32)
    zs = zs + jnp.dot(ps0_ref[...] + ps1_ref[...], wb,
                      preferred_element_type=jnp.float32)
    zs = jnp.maximum(zs + bl, 0.0)
    ns = jnp.sqrt(jnp.sum(zs * zs, axis=1, keepdims=True))
    zs = zs / jnp.maximum(ns, 1e-12)

    t = za * jnp.dot(zs, w4_ref[...], preferred_element_type=jnp.float32)
    q_ref[...] = jnp.dot(jnp.maximum(t, 0.0), w5_ref[...],
                         preferred_element_type=jnp.float32)


def _decode(ah, ap0, ap1, hs, ps0, ps1, wa, wb, bl2, w4, w5):
    b = ah.shape[0]
    return pl.pallas_call(
        _dec_body,
        out_shape=jax.ShapeDtypeStruct((b, 1), jnp.float32),
    )(ah, ap0, ap1, hs, ps0, ps1, wa, wb, bl2, w4, w5)


_K = 80


@functools.lru_cache(maxsize=None)
def _make_sc_agg(n, emb, e, b, m):
    info = plsc.get_sparse_core_info()
    nc, ns = info.num_cores, info.num_subcores
    nw = nc * ns
    epw = e // nw
    mpw = m // nw
    bpw = b // nw
    bps = b // ns
    rpt = 8 * _K
    np_ = ns * rpt
    assert epw % _K == 0 and mpw % _K == 0 and np_ >= n
    assert b % (8 * nw) == 0 and bps % 8 == 0

    mesh = plsc.VectorSubcoreMesh(core_axis_name="c", subcore_axis_name="s")

    @functools.partial(
        pl.kernel,
        out_type=[
            jax.ShapeDtypeStruct((np_, emb), jnp.float32),
            jax.ShapeDtypeStruct((np_, emb), jnp.float32),
            jax.ShapeDtypeStruct((b, emb), jnp.float32),
            jax.ShapeDtypeStruct((b, emb), jnp.float32),
            jax.ShapeDtypeStruct((b, emb), jnp.float32),
            jax.ShapeDtypeStruct((b, emb), jnp.float32),
            jax.ShapeDtypeStruct((b, emb), jnp.float32),
        ],
        mesh=mesh,
        scratch_types=[
            pltpu.VMEM_SHARED((np_, emb), jnp.float32),
            pltpu.VMEM_SHARED((b, emb), jnp.float32),
            pltpu.VMEM((_K, emb), jnp.float32),
            pltpu.VMEM((_K,), jnp.int32),
            pltpu.VMEM((_K,), jnp.int32),
            pltpu.VMEM((_K, emb), jnp.float32),
            pltpu.VMEM((bpw,), jnp.int32),
            pltpu.VMEM((bpw, emb), jnp.float32),
            pltpu.VMEM((bps,), jnp.int32),
            pltpu.VMEM((bps, emb), jnp.float32),
            pltpu.SemaphoreType.DMA,
        ],
    )
    def sc_agg(h, row, col, bnr, bnc, act,
               p0, p1, ps0, ps1, act_h, act_p0, act_p1,
               acc, acc_s, zbuf, idxa, idxb, rows_v, aidx, abuf,
               aidx2, abuf2, sem):
        c = lax.axis_index("c")
        s = lax.axis_index("s")
        wid = s * nc + c

        def zb(i, carry):
            for j in range(emb // 16):
                zbuf[i, pl.ds(j * 16, 16)] = jnp.zeros((16,), jnp.float32)
            return carry
        lax.fori_loop(0, _K, zb, 0)
        for j in range(rpt // _K):
            pltpu.sync_copy(zbuf, acc.at[pl.ds(s * rpt + j * _K, _K)])
        pltpu.sync_copy(zbuf.at[pl.ds(0, b // ns)],
                        acc_s.at[pl.ds(s * (b // ns), b // ns)])
        plsc.subcore_barrier()

        ebase = wid * epw

        def edge_step(i, carry):
            base = pl.multiple_of(ebase + i * _K, 8)
            pltpu.sync_copy(col.at[pl.ds(base, _K)], idxa)
            pltpu.async_copy(h.at[idxa], rows_v, sem).wait()
            pltpu.sync_copy(row.at[pl.ds(base, _K)], idxb)
            pltpu.sync_copy(rows_v, acc.at[idxb], add=True)
            return carry
        lax.fori_loop(0, epw // _K, edge_step, 0)

        mbase = wid * mpw

        def bn_step(i, carry):
            base = pl.multiple_of(mbase + i * _K, 8)
            pltpu.sync_copy(bnc.at[pl.ds(base, _K)], idxa)
            pltpu.async_copy(h.at[idxa], rows_v, sem).wait()
            pltpu.sync_copy(bnr.at[pl.ds(base, _K)], idxb)
            pltpu.sync_copy(rows_v, acc_s.at[idxb], add=True)
            return carry
        lax.fori_loop(0, mpw // _K, bn_step, 0)

        abase = wid * bpw
        pltpu.sync_copy(act.at[pl.ds(abase, bpw)], aidx)
        pltpu.async_copy(h.at[aidx], abuf, sem).wait()
        pltpu.sync_copy(abuf, act_h.at[pl.ds(abase, bpw)])

        plsc.subcore_barrier()

        @pl.when(c == 0)
        def _():
            pltpu.sync_copy(acc.at[pl.ds(s * rpt, rpt)],
                            p0.at[pl.ds(s * rpt, rpt)])
            pltpu.sync_copy(acc_s.at[pl.ds(s * (b // ns), b // ns)],
                            ps0.at[pl.ds(s * (b // ns), b // ns)])

        @pl.when(c == 1)
        def _():
            pltpu.sync_copy(acc.at[pl.ds(s * rpt, rpt)],
                            p1.at[pl.ds(s * rpt, rpt)])
            pltpu.sync_copy(acc_s.at[pl.ds(s * (b // ns), b // ns)],
                            ps1.at[pl.ds(s * (b // ns), b // ns)])

        plsc.subcore_barrier()

        b2 = s * bps
        pltpu.sync_copy(act.at[pl.ds(b2, bps)], aidx2)

        @pl.when(c == 0)
        def _():
            pltpu.async_copy(p0.at[aidx2], abuf2, sem).wait()
            pltpu.sync_copy(abuf2, act_p0.at[pl.ds(b2, bps)])

        @pl.when(c == 1)
        def _():
            pltpu.async_copy(p1.at[aidx2], abuf2, sem).wait()
            pltpu.sync_copy(abuf2, act_p1.at[pl.ds(b2, bps)])

    return sc_agg


def kernel(action_idx, edge_index, bn_row, bn_col, need_q_for_all,
           input_features, input_feature_s,
           W1, W2, W3, Wl, bl, W4, W5):
    n, d = input_features.shape
    emb = W1.shape[1]
    e = edge_index.shape[1]
    b = input_feature_s.shape[0]
    m = bn_row.shape[0]

    row = edge_index[0]
    col = edge_index[1]
    bl2 = bl.reshape(1, emb)

    sc_agg = _make_sc_agg(n, emb, e, b, m)

    h0 = _encode(input_features, W1, 400)
    h0s = _encode(input_feature_s, W1, b)
    wa, wb = _fold_weights(W2, W3, Wl)

    p0, p1, ps0, ps1, _, _, _ = sc_agg(h0, row, col, bn_row, bn_col,
                                       action_idx)
    h1 = _layer(h0, p0, p1, wa, wb, bl2, 400)
    h1s = _layer(h0s, ps0, ps1, wa, wb, bl2, b)

    _, _, qs0, qs1, ah, ap0, ap1 = sc_agg(h1, row, col, bn_row, bn_col,
                                          action_idx)
    return _decode(ah, ap0, ap1, h1s, qs0, qs1, wa, wb, bl2, W4, W5)


# speedup vs baseline: 11.5018x; 2.5703x over previous
import functools

import jax
import jax.numpy as jnp
from jax import lax
from jax.experimental import pallas as pl
from jax.experimental.pallas import tpu as pltpu
from jax.experimental.pallas import tpu_sc as plsc


def _enc_body(x_ref, w_ref, o_ref):
    r = jnp.dot(x_ref[...], w_ref[...], preferred_element_type=jnp.float32)
    r = jnp.maximum(r, 0.0)
    n = jnp.sqrt(jnp.sum(r * r, axis=1, keepdims=True))
    o_ref[...] = r / jnp.maximum(n, 1e-12)


def _encode(x, w, blk):
    n, d = x.shape
    emb = w.shape[1]
    return pl.pallas_call(
        _enc_body,
        grid=(n // blk,),
        in_specs=[
            pl.BlockSpec((blk, d), lambda i: (i, 0)),
            pl.BlockSpec((d, emb), lambda i: (0, 0)),
        ],
        out_specs=pl.BlockSpec((blk, emb), lambda i: (i, 0)),
        out_shape=jax.ShapeDtypeStruct((n, emb), jnp.float32),
    )(x, w)


def _wab_body(w2_ref, w3_ref, wl_ref, wa_ref, wb_ref):
    emb = w2_ref.shape[0]
    wl = wl_ref[...]
    dn = (((1,), (1,)), ((), ()))
    wa_ref[...] = lax.dot_general(
        w2_ref[...], wl[:, :emb], dn, preferred_element_type=jnp.float32)
    wb_ref[...] = lax.dot_general(
        w3_ref[...], wl[:, emb:], dn, preferred_element_type=jnp.float32)


def _fold_weights(w2, w3, wl):
    emb = w2.shape[0]
    return pl.pallas_call(
        _wab_body,
        out_shape=(
            jax.ShapeDtypeStruct((emb, emb), jnp.float32),
            jax.ShapeDtypeStruct((emb, emb), jnp.float32),
        ),
    )(w2, w3, wl)


def _layer_body(h_ref, p0_ref, p1_ref, wa_ref, wb_ref, bl_ref, o_ref):
    agg = p0_ref[...] + p1_ref[...]
    y = jnp.dot(h_ref[...], wa_ref[...], preferred_element_type=jnp.float32)
    y = y + jnp.dot(agg, wb_ref[...], preferred_element_type=jnp.float32)
    y = jnp.maximum(y + bl_ref[...], 0.0)
    n = jnp.sqrt(jnp.sum(y * y, axis=1, keepdims=True))
    o_ref[...] = y / jnp.maximum(n, 1e-12)


def _layer(h, p0, p1, wa, wb, bl2, blk):
    n, emb = h.shape
    return pl.pallas_call(
        _layer_body,
        grid=(n // blk,),
        in_specs=[
            pl.BlockSpec((blk, emb), lambda i: (i, 0)),
            pl.BlockSpec((blk, emb), lambda i: (i, 0)),
            pl.BlockSpec((blk, emb), lambda i: (i, 0)),
            pl.BlockSpec((emb, emb), lambda i: (0, 0)),
            pl.BlockSpec((emb, emb), lambda i: (0, 0)),
            pl.BlockSpec((1, emb), lambda i: (0, 0)),
        ],
        out_specs=pl.BlockSpec((blk, emb), lambda i: (i, 0)),
        out_shape=jax.ShapeDtypeStruct((n, emb), jnp.float32),
    )(h, p0, p1, wa, wb, bl2)


def _dec_body(ah_ref, ap0_ref, ap1_ref, hs_ref, ps0_ref, ps1_ref,
              wa_ref, wb_ref, bl_ref, w4_ref, w5_ref, q_ref):
    wa = wa_ref[...]
    wb = wb_ref[...]
    bl = bl_ref[...]

    za = jnp.dot(ah_ref[...], wa, preferred_element_type=jnp.float32)
    za = za + jnp.dot(ap0_ref[...] + ap1_ref[...], wb,
                      preferred_element_type=jnp.float32)
    za = jnp.maximum(za + bl, 0.0)
    na = jnp.sqrt(jnp.sum(za * za, axis=1, keepdims=True))
    za = za / jnp.maximum(na, 1e-12)

    zs = jnp.dot(hs_ref[...], wa, preferred_element_type=jnp.float32)
    zs = zs + jnp.dot(ps0_ref[...] + ps1_ref[...], wb,
                      preferred_element_type=jnp.float32)
    zs = jnp.maximum(zs + bl, 0.0)
    ns = jnp.sqrt(jnp.sum(zs * zs, axis=1, keepdims=True))
    zs = zs / jnp.maximum(ns, 1e-12)

    t = za * jnp.dot(zs, w4_ref[...], preferred_element_type=jnp.float32)
    q_ref[...] = jnp.dot(jnp.maximum(t, 0.0), w5_ref[...],
                         preferred_element_type=jnp.float32)


def _decode(ah, ap0, ap1, hs, ps0, ps1, wa, wb, bl2, w4, w5):
    b = ah.shape[0]
    return pl.pallas_call(
        _dec_body,
        out_shape=jax.ShapeDtypeStruct((b, 1), jnp.float32),
    )(ah, ap0, ap1, hs, ps0, ps1, wa, wb, bl2, w4, w5)


_K = 80
_NBUF = 3


@functools.lru_cache(maxsize=None)
def _make_sc_agg(n, emb, e, b, m):
    info = plsc.get_sparse_core_info()
    nc, ns = info.num_cores, info.num_subcores
    nw = nc * ns
    epw = e // nw
    mpw = m // nw
    bpw = b // nw
    bps = b // ns
    nec = epw // _K
    nmc = mpw // _K
    rpt = 8 * _K
    np_ = ns * rpt
    assert epw % _K == 0 and mpw % _K == 0 and np_ >= n
    assert b % (8 * nw) == 0 and bps % 8 == 0
    assert nec > 2 * _NBUF

    mesh = plsc.VectorSubcoreMesh(core_axis_name="c", subcore_axis_name="s")

    @functools.partial(
        pl.kernel,
        out_type=[
            jax.ShapeDtypeStruct((np_, emb), jnp.float32),
            jax.ShapeDtypeStruct((np_, emb), jnp.float32),
            jax.ShapeDtypeStruct((b, emb), jnp.float32),
            jax.ShapeDtypeStruct((b, emb), jnp.float32),
            jax.ShapeDtypeStruct((b, emb), jnp.float32),
            jax.ShapeDtypeStruct((b, emb), jnp.float32),
            jax.ShapeDtypeStruct((b, emb), jnp.float32),
        ],
        mesh=mesh,
        scratch_types=[
            pltpu.VMEM_SHARED((np_, emb), jnp.float32),
            pltpu.VMEM_SHARED((b, emb), jnp.float32),
            pltpu.VMEM((epw,), jnp.int32),
            [pltpu.VMEM((_K,), jnp.int32)] * _NBUF,
            [pltpu.VMEM((_K,), jnp.int32)] * _NBUF,
            [pltpu.VMEM((_K, emb), jnp.float32)] * _NBUF,
            pltpu.VMEM((bpw,), jnp.int32),
            pltpu.VMEM((bps,), jnp.int32),
            [pltpu.SemaphoreType.DMA] * _NBUF,
            pltpu.SemaphoreType.DMA,
        ],
    )
    def sc_agg(h, row, col, bnr, bnc, act,
               p0, p1, ps0, ps1, act_h, act_p0, act_p1,
               acc, acc_s, cidx, cbuf, idxr, rows, aidx, aidx2, gsem, sem):
        c = lax.axis_index("c")
        s = lax.axis_index("s")
        wid = s * nc + c

        pltpu.sync_copy(col.at[pl.ds(wid * epw, epw)], cidx)

        def fill_cbuf(ch, bidx):
            for j in range(_K // 16):
                off = pl.multiple_of(ch * _K, 16) + j * 16
                cbuf[bidx][pl.ds(j * 16, 16)] = cidx[pl.ds(off, 16)]

        def zb(i, carry):
            for j in range(emb // 16):
                rows[0][i, pl.ds(j * 16, 16)] = jnp.zeros((16,), jnp.float32)
            return carry
        lax.fori_loop(0, _K, zb, 0)
        for j in range(rpt // _K):
            pltpu.sync_copy(rows[0], acc.at[pl.ds(s * rpt + j * _K, _K)])
        pltpu.sync_copy(rows[0].at[pl.ds(0, b // ns)],
                        acc_s.at[pl.ds(s * (b // ns), b // ns)])
        plsc.subcore_barrier()

        def wait_slot(bidx):
            pltpu.make_async_copy(h.at[cbuf[bidx]], rows[bidx],
                                  gsem[bidx]).wait()
            pltpu.make_async_copy(row.at[pl.ds(0, _K)], idxr[bidx],
                                  gsem[bidx]).wait()

        ebase = wid * epw

        def issue_edge(ch, bidx):
            base = pl.multiple_of(ebase + ch * _K, 8)
            fill_cbuf(ch, bidx)
            pltpu.async_copy(row.at[pl.ds(base, _K)], idxr[bidx], gsem[bidx])
            pltpu.async_copy(h.at[cbuf[bidx]], rows[bidx], gsem[bidx])

        for bidx in range(_NBUF):
            issue_edge(bidx, bidx)

        nfull = (nec - _NBUF) // _NBUF

        def edge_turn(j, carry):
            for bidx in range(_NBUF):
                ch = j * _NBUF + bidx
                wait_slot(bidx)
                pltpu.sync_copy(rows[bidx], acc.at[idxr[bidx]], add=True)
                issue_edge(ch + _NBUF, bidx)
            return carry
        lax.fori_loop(0, nfull, edge_turn, 0)

        for ch in range(nfull * _NBUF, nec):
            bidx = ch % _NBUF
            wait_slot(bidx)
            pltpu.sync_copy(rows[bidx], acc.at[idxr[bidx]], add=True)
            if ch + _NBUF < nec:
                issue_edge(ch + _NBUF, bidx)

        pltpu.sync_copy(bnc.at[pl.ds(wid * mpw, mpw)],
                        cidx.at[pl.ds(0, mpw)])
        mbase = wid * mpw

        def issue_bn(ch, bidx):
            base = pl.multiple_of(mbase + ch * _K, 8)
            fill_cbuf(ch, bidx)
            pltpu.async_copy(bnr.at[pl.ds(base, _K)], idxr[bidx], gsem[bidx])
            pltpu.async_copy(h.at[cbuf[bidx]], rows[bidx], gsem[bidx])

        for ch in range(min(nmc, _NBUF)):
            issue_bn(ch, ch)
        for ch in range(nmc):
            bidx = ch % _NBUF
            wait_slot(bidx)
            pltpu.sync_copy(rows[bidx], acc_s.at[idxr[bidx]], add=True)
            if ch + _NBUF < nmc:
                issue_bn(ch + _NBUF, bidx)

        abase = wid * bpw
        abuf = rows[1].at[pl.ds(0, bpw)]
        pltpu.sync_copy(act.at[pl.ds(abase, bpw)], aidx)
        pltpu.async_copy(h.at[aidx], abuf, sem).wait()
        pltpu.sync_copy(abuf, act_h.at[pl.ds(abase, bpw)])

        plsc.subcore_barrier()

        @pl.when(c == 0)
        def _():
            pltpu.sync_copy(acc.at[pl.ds(s * rpt, rpt)],
                            p0.at[pl.ds(s * rpt, rpt)])
            pltpu.sync_copy(acc_s.at[pl.ds(s * (b // ns), b // ns)],
                            ps0.at[pl.ds(s * (b // ns), b // ns)])

        @pl.when(c == 1)
        def _():
            pltpu.sync_copy(acc.at[pl.ds(s * rpt, rpt)],
                            p1.at[pl.ds(s * rpt, rpt)])
            pltpu.sync_copy(acc_s.at[pl.ds(s * (b // ns), b // ns)],
                            ps1.at[pl.ds(s * (b // ns), b // ns)])

        plsc.subcore_barrier()

        b2 = s * bps
        abuf2 = rows[2].at[pl.ds(0, bps)]
        pltpu.sync_copy(act.at[pl.ds(b2, bps)], aidx2)

        @pl.when(c == 0)
        def _():
            pltpu.async_copy(p0.at[aidx2], abuf2, sem).wait()
            pltpu.sync_copy(abuf2, act_p0.at[pl.ds(b2, bps)])

        @pl.when(c == 1)
        def _():
            pltpu.async_copy(p1.at[aidx2], abuf2, sem).wait()
            pltpu.sync_copy(abuf2, act_p1.at[pl.ds(b2, bps)])

    return sc_agg


def kernel(action_idx, edge_index, bn_row, bn_col, need_q_for_all,
           input_features, input_feature_s,
           W1, W2, W3, Wl, bl, W4, W5):
    n, d = input_features.shape
    emb = W1.shape[1]
    e = edge_index.shape[1]
    b = input_feature_s.shape[0]
    m = bn_row.shape[0]

    row = edge_index[0]
    col = edge_index[1]
    bl2 = bl.reshape(1, emb)

    sc_agg = _make_sc_agg(n, emb, e, b, m)

    h0 = _encode(input_features, W1, 400)
    h0s = _encode(input_feature_s, W1, b)
    wa, wb = _fold_weights(W2, W3, Wl)

    p0, p1, ps0, ps1, _, _, _ = sc_agg(h0, row, col, bn_row, bn_col,
                                       action_idx)
    h1 = _layer(h0, p0, p1, wa, wb, bl2, 400)
    h1s = _layer(h0s, ps0, ps1, wa, wb, bl2, b)

    _, _, qs0, qs1, ah, ap0, ap1 = sc_agg(h1, row, col, bn_row, bn_col,
                                          action_idx)
    return _decode(ah, ap0, ap1, h1s, qs0, qs1, wa, wb, bl2, W4, W5)


# speedup vs baseline: 11.6535x; 1.0132x over previous
import functools

import jax
import jax.numpy as jnp
from jax import lax
from jax.experimental import pallas as pl
from jax.experimental.pallas import tpu as pltpu
from jax.experimental.pallas import tpu_sc as plsc


def _enc_body(x_ref, w_ref, o_ref):
    r = jnp.dot(x_ref[...], w_ref[...], preferred_element_type=jnp.float32)
    r = jnp.maximum(r, 0.0)
    n = jnp.sqrt(jnp.sum(r * r, axis=1, keepdims=True))
    o_ref[...] = r / jnp.maximum(n, 1e-12)


def _encode(x, w, blk):
    n, d = x.shape
    emb = w.shape[1]
    return pl.pallas_call(
        _enc_body,
        grid=(n // blk,),
        in_specs=[
            pl.BlockSpec((blk, d), lambda i: (i, 0)),
            pl.BlockSpec((d, emb), lambda i: (0, 0)),
        ],
        out_specs=pl.BlockSpec((blk, emb), lambda i: (i, 0)),
        out_shape=jax.ShapeDtypeStruct((n, emb), jnp.float32),
    )(x, w)


def _wab_body(w2_ref, w3_ref, wl_ref, xs_ref, w1_ref,
              wa_ref, wb_ref, hs_ref):
    emb = w2_ref.shape[0]
    wl = wl_ref[...]
    dn = (((1,), (1,)), ((), ()))
    wa_ref[...] = lax.dot_general(
        w2_ref[...], wl[:, :emb], dn, preferred_element_type=jnp.float32)
    wb_ref[...] = lax.dot_general(
        w3_ref[...], wl[:, emb:], dn, preferred_element_type=jnp.float32)
    _enc_body(xs_ref, w1_ref, hs_ref)


def _fold_weights(w2, w3, wl, xs, w1):
    emb = w2.shape[0]
    return pl.pallas_call(
        _wab_body,
        out_shape=(
            jax.ShapeDtypeStruct((emb, emb), jnp.float32),
            jax.ShapeDtypeStruct((emb, emb), jnp.float32),
            jax.ShapeDtypeStruct((xs.shape[0], emb), jnp.float32),
        ),
    )(w2, w3, wl, xs, w1)


def _layer_body(h_ref, p0_ref, p1_ref, wa_ref, wb_ref, bl_ref, o_ref):
    agg = p0_ref[...] + p1_ref[...]
    y = jnp.dot(h_ref[...], wa_ref[...], preferred_element_type=jnp.float32)
    y = y + jnp.dot(agg, wb_ref[...], preferred_element_type=jnp.float32)
    y = jnp.maximum(y + bl_ref[...], 0.0)
    n = jnp.sqrt(jnp.sum(y * y, axis=1, keepdims=True))
    o_ref[...] = y / jnp.maximum(n, 1e-12)


def _layer(h, p0, p1, wa, wb, bl2, blk):
    n, emb = h.shape
    return pl.pallas_call(
        _layer_body,
        grid=(n // blk,),
        in_specs=[
            pl.BlockSpec((blk, emb), lambda i: (i, 0)),
            pl.BlockSpec((blk, emb), lambda i: (i, 0)),
            pl.BlockSpec((blk, emb), lambda i: (i, 0)),
            pl.BlockSpec((emb, emb), lambda i: (0, 0)),
            pl.BlockSpec((emb, emb), lambda i: (0, 0)),
            pl.BlockSpec((1, emb), lambda i: (0, 0)),
        ],
        out_specs=pl.BlockSpec((blk, emb), lambda i: (i, 0)),
        out_shape=jax.ShapeDtypeStruct((n, emb), jnp.float32),
    )(h, p0, p1, wa, wb, bl2)


def _dec_body(ah_ref, ap0_ref, ap1_ref, h0s_ref, ps0_ref, ps1_ref,
              qs0_ref, qs1_ref, wa_ref, wb_ref, bl_ref, w4_ref, w5_ref,
              q_ref):
    wa = wa_ref[...]
    wb = wb_ref[...]
    bl = bl_ref[...]

    def xf(hv, agg):
        y = jnp.dot(hv, wa, preferred_element_type=jnp.float32)
        y = y + jnp.dot(agg, wb, preferred_element_type=jnp.float32)
        y = jnp.maximum(y + bl, 0.0)
        n = jnp.sqrt(jnp.sum(y * y, axis=1, keepdims=True))
        return y / jnp.maximum(n, 1e-12)

    za = xf(ah_ref[...], ap0_ref[...] + ap1_ref[...])
    h1s = xf(h0s_ref[...], ps0_ref[...] + ps1_ref[...])
    zs = xf(h1s, qs0_ref[...] + qs1_ref[...])

    t = za * jnp.dot(zs, w4_ref[...], preferred_element_type=jnp.float32)
    q_ref[...] = jnp.dot(jnp.maximum(t, 0.0), w5_ref[...],
                         preferred_element_type=jnp.float32)


def _decode(ah, ap0, ap1, h0s, ps0, ps1, qs0, qs1, wa, wb, bl2, w4, w5):
    b = ah.shape[0]
    return pl.pallas_call(
        _dec_body,
        out_shape=jax.ShapeDtypeStruct((b, 1), jnp.float32),
    )(ah, ap0, ap1, h0s, ps0, ps1, qs0, qs1, wa, wb, bl2, w4, w5)


_K = 80
_NBUF = 3


@functools.lru_cache(maxsize=None)
def _make_sc_agg(n, emb, e, b, m):
    info = plsc.get_sparse_core_info()
    nc, ns = info.num_cores, info.num_subcores
    nw = nc * ns
    epw = e // nw
    mpw = m // nw
    bpw = b // nw
    bps = b // ns
    nec = epw // _K
    nmc = mpw // _K
    rpt = 8 * _K
    np_ = ns * rpt
    assert epw % _K == 0 and mpw % _K == 0 and np_ >= n
    assert b % (8 * nw) == 0 and bps % 8 == 0
    assert nec > 2 * _NBUF

    mesh = plsc.VectorSubcoreMesh(core_axis_name="c", subcore_axis_name="s")

    @functools.partial(
        pl.kernel,
        out_type=[
            jax.ShapeDtypeStruct((np_, emb), jnp.float32),
            jax.ShapeDtypeStruct((np_, emb), jnp.float32),
            jax.ShapeDtypeStruct((b, emb), jnp.float32),
            jax.ShapeDtypeStruct((b, emb), jnp.float32),
            jax.ShapeDtypeStruct((b, emb), jnp.float32),
            jax.ShapeDtypeStruct((b, emb), jnp.float32),
            jax.ShapeDtypeStruct((b, emb), jnp.float32),
        ],
        mesh=mesh,
        scratch_types=[
            pltpu.VMEM_SHARED((np_, emb), jnp.float32),
            pltpu.VMEM_SHARED((b, emb), jnp.float32),
            pltpu.VMEM((16, emb), jnp.float32),
            pltpu.VMEM((epw,), jnp.int32),
            [pltpu.VMEM((_K,), jnp.int32)] * _NBUF,
            [pltpu.VMEM((_K,), jnp.int32)] * _NBUF,
            [pltpu.VMEM((_K, emb), jnp.float32)] * _NBUF,
            pltpu.VMEM((bpw,), jnp.int32),
            pltpu.VMEM((bps,), jnp.int32),
            [pltpu.SemaphoreType.DMA] * _NBUF,
            pltpu.SemaphoreType.DMA,
        ],
    )
    def sc_agg(h, row, col, bnr, bnc, act,
               p0, p1, ps0, ps1, act_h, act_p0, act_p1,
               acc, acc_s, zbuf, cidx, cbuf, idxr, rows, aidx, aidx2,
               gsem, sem):
        c = lax.axis_index("c")
        s = lax.axis_index("s")
        wid = s * nc + c

        pltpu.sync_copy(col.at[pl.ds(wid * epw, epw)], cidx)

        def fill_cbuf(ch, bidx):
            for j in range(_K // 16):
                off = pl.multiple_of(ch * _K, 16) + j * 16
                cbuf[bidx][pl.ds(j * 16, 16)] = cidx[pl.ds(off, 16)]

        def wait_slot(bidx):
            pltpu.make_async_copy(h.at[cbuf[bidx]], rows[bidx],
                                  gsem[bidx]).wait()
            pltpu.make_async_copy(row.at[pl.ds(0, _K)], idxr[bidx],
                                  gsem[bidx]).wait()

        ebase = wid * epw

        def issue_edge(ch, bidx):
            base = pl.multiple_of(ebase + ch * _K, 8)
            fill_cbuf(ch, bidx)
            pltpu.async_copy(row.at[pl.ds(base, _K)], idxr[bidx], gsem[bidx])
            pltpu.async_copy(h.at[cbuf[bidx]], rows[bidx], gsem[bidx])

        for bidx in range(_NBUF):
            issue_edge(bidx, bidx)

        def zb(i, carry):
            for j in range(emb // 16):
                zbuf[i, pl.ds(j * 16, 16)] = jnp.zeros((16,), jnp.float32)
            return carry
        lax.fori_loop(0, 16, zb, 0)

        def zcp(j, carry):
            pltpu.sync_copy(zbuf, acc.at[pl.ds(s * rpt + j * 16, 16)])
            return carry
        lax.fori_loop(0, rpt // 16, zcp, 0)
        pltpu.sync_copy(zbuf, acc_s.at[pl.ds(s * (b // ns), b // ns)])
        plsc.subcore_barrier()

        nfull = (nec - _NBUF) // _NBUF

        def edge_turn(j, carry):
            for bidx in range(_NBUF):
                ch = j * _NBUF + bidx
                wait_slot(bidx)
                pltpu.sync_copy(rows[bidx], acc.at[idxr[bidx]], add=True)
                issue_edge(ch + _NBUF, bidx)
            return carry
        lax.fori_loop(0, nfull, edge_turn, 0)

        for ch in range(nfull * _NBUF, nec):
            bidx = ch % _NBUF
            wait_slot(bidx)
            pltpu.sync_copy(rows[bidx], acc.at[idxr[bidx]], add=True)
            if ch + _NBUF < nec:
                issue_edge(ch + _NBUF, bidx)

        pltpu.sync_copy(bnc.at[pl.ds(wid * mpw, mpw)],
                        cidx.at[pl.ds(0, mpw)])
        mbase = wid * mpw

        def issue_bn(ch, bidx):
            base = pl.multiple_of(mbase + ch * _K, 8)
            fill_cbuf(ch, bidx)
            pltpu.async_copy(bnr.at[pl.ds(base, _K)], idxr[bidx], gsem[bidx])
            pltpu.async_copy(h.at[cbuf[bidx]], rows[bidx], gsem[bidx])

        for ch in range(min(nmc, _NBUF)):
            issue_bn(ch, ch)
        for ch in range(nmc):
            bidx = ch % _NBUF
            wait_slot(bidx)
            pltpu.sync_copy(rows[bidx], acc_s.at[idxr[bidx]], add=True)
            if ch + _NBUF < nmc:
                issue_bn(ch + _NBUF, bidx)

        abase = wid * bpw
        abuf = rows[1].at[pl.ds(0, bpw)]
        pltpu.sync_copy(act.at[pl.ds(abase, bpw)], aidx)
        pltpu.async_copy(h.at[aidx], abuf, sem).wait()
        pltpu.sync_copy(abuf, act_h.at[pl.ds(abase, bpw)])

        plsc.subcore_barrier()

        @pl.when(c == 0)
        def _():
            pltpu.sync_copy(acc.at[pl.ds(s * rpt, rpt)],
                            p0.at[pl.ds(s * rpt, rpt)])
            pltpu.sync_copy(acc_s.at[pl.ds(s * (b // ns), b // ns)],
                            ps0.at[pl.ds(s * (b // ns), b // ns)])

        @pl.when(c == 1)
        def _():
            pltpu.sync_copy(acc.at[pl.ds(s * rpt, rpt)],
                            p1.at[pl.ds(s * rpt, rpt)])
            pltpu.sync_copy(acc_s.at[pl.ds(s * (b // ns), b // ns)],
                            ps1.at[pl.ds(s * (b // ns), b // ns)])

        plsc.subcore_barrier()

        b2 = s * bps
        abuf2 = rows[2].at[pl.ds(0, bps)]
        pltpu.sync_copy(act.at[pl.ds(b2, bps)], aidx2)

        @pl.when(c == 0)
        def _():
            pltpu.async_copy(p0.at[aidx2], abuf2, sem).wait()
            pltpu.sync_copy(abuf2, act_p0.at[pl.ds(b2, bps)])

        @pl.when(c == 1)
        def _():
            pltpu.async_copy(p1.at[aidx2], abuf2, sem).wait()
            pltpu.sync_copy(abuf2, act_p1.at[pl.ds(b2, bps)])

    return sc_agg


def kernel(action_idx, edge_index, bn_row, bn_col, need_q_for_all,
           input_features, input_feature_s,
           W1, W2, W3, Wl, bl, W4, W5):
    n, d = input_features.shape
    emb = W1.shape[1]
    e = edge_index.shape[1]
    b = input_feature_s.shape[0]
    m = bn_row.shape[0]

    row = edge_index[0]
    col = edge_index[1]
    bl2 = bl.reshape(1, emb)

    sc_agg = _make_sc_agg(n, emb, e, b, m)

    h0 = _encode(input_features, W1, 400)
    wa, wb, h0s = _fold_weights(W2, W3, Wl, input_feature_s, W1)

    p0, p1, ps0, ps1, _, _, _ = sc_agg(h0, row, col, bn_row, bn_col,
                                       action_idx)
    h1 = _layer(h0, p0, p1, wa, wb, bl2, 400)

    _, _, qs0, qs1, ah, ap0, ap1 = sc_agg(h1, row, col, bn_row, bn_col,
                                          action_idx)
    return _decode(ah, ap0, ap1, h0s, ps0, ps1, qs0, qs1, wa, wb, bl2,
                   W4, W5)
